# Initial kernel scaffold; baseline (speedup 1.0000x reference)
import functools

import jax
import jax.numpy as jnp
from jax import lax
from jax.experimental import pallas as pl
from jax.experimental.pallas import tpu as pltpu
from jax.experimental.pallas import tpu_sc as plsc

N = 10000
E = 320000
D = 128
H = 128
WPAD = 144
NC = 2
NS = 16
NW = NC * NS
EW = E // NW
CH = 80
NCHUNK = EW // CH
RPT = N // NS
ZR = 125

_MESH = plsc.VectorSubcoreMesh(
    core_axis_name="c", subcore_axis_name="s", num_cores=NC, num_subcores=NS)


def _make_sc_scatter(W):
    def body(table, src, dst, out0, out1, idx_s, idx_d, gbuf, zbuf, acc):
        c = lax.axis_index("c")
        s = lax.axis_index("s")
        wid = c * NS + s

        def zrow(r, _):
            for cb in range(W // 16):
                zbuf[r, pl.ds(cb * 16, 16)] = jnp.zeros((16,), jnp.float32)
            return 0
        lax.fori_loop(0, ZR, zrow, 0)
        for kz in range(RPT // ZR):
            pltpu.sync_copy(zbuf, acc.at[pl.ds(s * RPT + kz * ZR, ZR)])
        plsc.subcore_barrier()

        def chunk(k, _):
            base = wid * EW + k * CH
            pltpu.sync_copy(src.at[pl.ds(base, CH)], idx_s)
            pltpu.sync_copy(dst.at[pl.ds(base, CH)], idx_d)
            pltpu.sync_copy(table.at[idx_s], gbuf)
            pltpu.sync_copy(gbuf, acc.at[idx_d], add=True)
            return 0
        lax.fori_loop(0, NCHUNK, chunk, 0)
        plsc.subcore_barrier()

        rows = pl.ds(s * RPT, RPT)

        @pl.when(c == 0)
        def _():
            pltpu.sync_copy(acc.at[rows], out0.at[rows])

        @pl.when(c == 1)
        def _():
            pltpu.sync_copy(acc.at[rows], out1.at[rows])

    sds = jax.ShapeDtypeStruct((N, W), jnp.float32)
    return pl.kernel(
        body,
        out_type=(sds, sds),
        mesh=_MESH,
        scratch_types=[
            pltpu.VMEM((CH,), jnp.int32),
            pltpu.VMEM((CH,), jnp.int32),
            pltpu.VMEM((CH, W), jnp.float32),
            pltpu.VMEM((ZR, W), jnp.float32),
            pltpu.VMEM_SHARED((N, W), jnp.float32),
        ],
    )


_sc_scatter_144 = _make_sc_scatter(WPAD)
_sc_scatter_128 = _make_sc_scatter(H)


def _edge_body(psrc, pdst, src, dst, ew, par, out,
               idx_s, idx_d, wbuf, bufS, bufD, pbuf, obuf):
    c = lax.axis_index("c")
    s = lax.axis_index("s")
    wid = c * NS + s
    pltpu.sync_copy(par, pbuf)
    lanes = lax.iota(jnp.int32, 16)

    def chunk(k, _):
        base = wid * EW + k * CH
        pltpu.sync_copy(src.at[pl.ds(base, CH)], idx_s)
        pltpu.sync_copy(dst.at[pl.ds(base, CH)], idx_d)
        pltpu.sync_copy(ew.at[pl.ds(base, CH)], wbuf)
        pltpu.sync_copy(psrc.at[idx_s], bufS)
        pltpu.sync_copy(pdst.at[idx_d], bufD)
        for g in range(CH // 16):
            rows = lanes + g * 16
            w16 = wbuf[pl.ds(g * 16, 16)]
            s16 = idx_s[pl.ds(g * 16, 16)]
            d16 = idx_d[pl.ds(g * 16, 16)]

            def jblk(jo, acc):
                for ju in range(8):
                    j = jo * 8 + ju
                    jb = jnp.full((16,), j, jnp.int32)
                    gs = plsc.load_gather(bufS, [rows, jb])
                    gd = plsc.load_gather(bufD, [rows, jb])
                    t = gs + gd + (w16 * pbuf[j] + pbuf[64 + j])
                    t = jnp.maximum(t, 0.0)
                    acc = acc + t * pbuf[128 + j]
                return acc
            acc = lax.fori_loop(0, 8, jblk, jnp.zeros((16,), jnp.float32))
            logit = acc + pbuf[192]
            logit = jnp.where(s16 == d16, jnp.float32(-1e9), logit)
            obuf[pl.ds(g * 16, 16)] = logit
        pltpu.sync_copy(obuf, out.at[pl.ds(base, CH)])
        return 0
    lax.fori_loop(0, NCHUNK, chunk, 0)


_sc_edge = pl.kernel(
    _edge_body,
    out_type=jax.ShapeDtypeStruct((E,), jnp.float32),
    mesh=_MESH,
    scratch_types=[
        pltpu.VMEM((CH,), jnp.int32),
        pltpu.VMEM((CH,), jnp.int32),
        pltpu.VMEM((CH,), jnp.float32),
        pltpu.VMEM((CH, 64), jnp.float32),
        pltpu.VMEM((CH, 64), jnp.float32),
        pltpu.VMEM((256,), jnp.float32),
        pltpu.VMEM((CH,), jnp.float32),
    ],
)


BR = 1000


def _tc1_body(x, p0, p1, w1s, w1n, b1, h1, deg):
    d = jnp.clip(p0[:, 128:129] + p1[:, 128:129], 1.0, None)
    agg = (p0[:, :128] + p1[:, :128]) / d
    h = x[...] @ w1s[...] + agg @ w1n[...] + b1[...]
    h1[...] = jnp.maximum(h, 0.0)
    deg[...] = d


def _tc1(x, p0, p1, w1s, w1n, b1):
    return pl.pallas_call(
        _tc1_body,
        grid=(N // BR,),
        in_specs=[
            pl.BlockSpec((BR, D), lambda i: (i, 0)),
            pl.BlockSpec((BR, WPAD), lambda i: (i, 0)),
            pl.BlockSpec((BR, WPAD), lambda i: (i, 0)),
            pl.BlockSpec((D, H), lambda i: (0, 0)),
            pl.BlockSpec((D, H), lambda i: (0, 0)),
            pl.BlockSpec((1, H), lambda i: (0, 0)),
        ],
        out_specs=[
            pl.BlockSpec((BR, H), lambda i: (i, 0)),
            pl.BlockSpec((BR, 1), lambda i: (i, 0)),
        ],
        out_shape=[
            jax.ShapeDtypeStruct((N, H), jnp.float32),
            jax.ShapeDtypeStruct((N, 1), jnp.float32),
        ],
    )(x, p0, p1, w1s, w1n, b1)


def _tc2_body(h1, q0, q1, deg, w2s, w2n, b2, we1a, we1b,
              wb1, bb1, wb2, bb2, wk1, bk1, wk2, bk2,
              psrc, pdst, bx, kc, ks):
    agg = (q0[...] + q1[...]) / deg[...]
    h2 = jnp.maximum(h1[...] @ w2s[...] + agg @ w2n[...] + b2[...], 0.0)
    psrc[...] = h2 @ we1a[...]
    pdst[...] = h2 @ we1b[...]
    tb = jnp.maximum(h2 @ wb1[...] + bb1[...], 0.0)
    bx[...] = tb @ wb2[...] + bb2[...]
    tk = jnp.maximum(h2 @ wk1[...] + bk1[...], 0.0)
    kl = tk @ wk2[...] + bk2[...]
    kcv = 1.0 + 7.0 * jax.nn.sigmoid(kl)
    kd = jnp.clip(jnp.round(kcv), 1.0, 8.0)
    kc[...] = kcv
    ks[...] = kcv + (kd - kcv)


def _tc2(h1, q0, q1, deg, pr):
    full = lambda a, b: pl.BlockSpec((a, b), lambda i: (0, 0))
    row = lambda b: pl.BlockSpec((BR, b), lambda i: (i, 0))
    return pl.pallas_call(
        _tc2_body,
        grid=(N // BR,),
        in_specs=[
            row(H), row(H), row(H), row(1),
            full(H, H), full(H, H), full(1, H),
            full(H, 64), full(H, 64),
            full(H, 64), full(1, 64), full(64, 2), full(1, 2),
            full(H, 32), full(1, 32), full(32, 1), full(1, 1),
        ],
        out_specs=[row(64), row(64), row(2), row(1), row(1)],
        out_shape=[
            jax.ShapeDtypeStruct((N, 64), jnp.float32),
            jax.ShapeDtypeStruct((N, 64), jnp.float32),
            jax.ShapeDtypeStruct((N, 2), jnp.float32),
            jax.ShapeDtypeStruct((N, 1), jnp.float32),
            jax.ShapeDtypeStruct((N, 1), jnp.float32),
        ],
    )(h1, q0, q1, deg, pr['W2s'], pr['W2n'], pr['b2'].reshape(1, H),
      pr['We1'][:H], pr['We1'][H:2 * H],
      pr['Wb1'], pr['bb1'].reshape(1, 64), pr['Wb2'], pr['bb2'].reshape(1, 2),
      pr['Wk1'], pr['bk1'].reshape(1, 32), pr['Wk2'], pr['bk2'].reshape(1, 1))


def kernel(x, edge_index, edge_weight, params):
    src = edge_index[0].astype(jnp.int32)
    dst = edge_index[1].astype(jnp.int32)
    xpad = jnp.concatenate(
        [x, jnp.ones((N, 1), jnp.float32), jnp.zeros((N, WPAD - D - 1), jnp.float32)],
        axis=1)

    p0, p1 = _sc_scatter_144(xpad, src, dst)
    h1, deg = _tc1(x, p0, p1, params['W1s'], params['W1n'],
                   params['b1'].reshape(1, H))
    q0, q1 = _sc_scatter_128(h1, src, dst)
    psrc, pdst, bx, kc, ks = _tc2(h1, q0, q1, deg, params)

    par = jnp.concatenate([
        params['We1'][2 * H],
        params['be1'],
        params['We2'][:, 0],
        jnp.broadcast_to(params['be2'], (64,)),
    ]).astype(jnp.float32)
    logits = _sc_edge(psrc, pdst, src, dst, edge_weight, par)

    return (logits, bx, kc[:, 0], ks[:, 0])


# speedup vs baseline: 3.1825x; 3.1825x over previous
import functools

import jax
import jax.numpy as jnp
from jax import lax
from jax.experimental import pallas as pl
from jax.experimental.pallas import tpu as pltpu
from jax.experimental.pallas import tpu_sc as plsc

N = 10000
E = 320000
D = 128
H = 128
WPAD = 144
NC = 2
NS = 16
NW = NC * NS
EW = E // NW
CH = 80
NCHUNK = EW // CH
RPT = N // NS
ZR = 125

_MESH = plsc.VectorSubcoreMesh(
    core_axis_name="c", subcore_axis_name="s", num_cores=NC, num_subcores=NS)
_SC_PARAMS = pltpu.CompilerParams(
    use_tc_tiling_on_sc=False, needs_layout_passes=False)


def _make_sc_scatter(W):
    def body(table, src, dst, out0, out1, idx_s, idx_d, gbuf, zbuf, acc):
        c = lax.axis_index("c")
        s = lax.axis_index("s")
        wid = c * NS + s

        def zrow(r, _):
            for cb in range(W // 16):
                zbuf[r, pl.ds(cb * 16, 16)] = jnp.zeros((16,), jnp.float32)
            return 0
        lax.fori_loop(0, ZR, zrow, 0)
        for kz in range(RPT // ZR):
            pltpu.sync_copy(zbuf, acc.at[pl.ds(s * RPT + kz * ZR, ZR)])
        plsc.subcore_barrier()

        def chunk(k, _):
            base = wid * EW + k * CH
            pltpu.sync_copy(src.at[pl.ds(base, CH)], idx_s)
            pltpu.sync_copy(dst.at[pl.ds(base, CH)], idx_d)
            pltpu.sync_copy(table.at[idx_s], gbuf)
            pltpu.sync_copy(gbuf, acc.at[idx_d], add=True)
            return 0
        lax.fori_loop(0, NCHUNK, chunk, 0)
        plsc.subcore_barrier()

        rows = pl.ds(s * RPT, RPT)

        @pl.when(c == 0)
        def _():
            pltpu.sync_copy(acc.at[rows], out0.at[rows])

        @pl.when(c == 1)
        def _():
            pltpu.sync_copy(acc.at[rows], out1.at[rows])

    sds = jax.ShapeDtypeStruct((N, W), jnp.float32)
    return pl.kernel(
        body,
        out_type=(sds, sds),
        mesh=_MESH,
        compiler_params=_SC_PARAMS,
        scratch_types=[
            pltpu.VMEM((CH,), jnp.int32),
            pltpu.VMEM((CH,), jnp.int32),
            pltpu.VMEM((CH, W), jnp.float32),
            pltpu.VMEM((ZR, W), jnp.float32),
            pltpu.VMEM_SHARED((N, W), jnp.float32),
        ],
    )


_sc_scatter_144 = _make_sc_scatter(WPAD)
_sc_scatter_128 = _make_sc_scatter(H)


def _edge_body(psrc, pdst, src, dst, ew, par, out,
               idx_s, idx_d, wbuf, bufS, bufD, pvbuf, pbuf, obuf):
    c = lax.axis_index("c")
    s = lax.axis_index("s")
    wid = c * NS + s
    pltpu.sync_copy(par, pvbuf)
    for i in range(16):
        v = pvbuf[pl.ds(i * 16, 16)]
        for l in range(16):
            pbuf[i * 16 + l] = v[l]
    lanes = lax.iota(jnp.int32, 16)

    def chunk(k, _):
        base = wid * EW + k * CH
        pltpu.sync_copy(src.at[pl.ds(base, CH)], idx_s)
        pltpu.sync_copy(dst.at[pl.ds(base, CH)], idx_d)
        pltpu.sync_copy(ew.at[pl.ds(base, CH)], wbuf)
        pltpu.sync_copy(psrc.at[idx_s], bufS)
        pltpu.sync_copy(pdst.at[idx_d], bufD)
        for g in range(CH // 16):
            rows = lanes + g * 16
            w16 = wbuf[pl.ds(g * 16, 16)]
            s16 = idx_s[pl.ds(g * 16, 16)]
            d16 = idx_d[pl.ds(g * 16, 16)]

            def jblk(jo, acc):
                for ju in range(8):
                    j = jo * 8 + ju
                    jb = jnp.full((16,), j, jnp.int32)
                    gs = plsc.load_gather(bufS, [rows, jb])
                    gd = plsc.load_gather(bufD, [rows, jb])
                    t = gs + gd + (w16 * pbuf[j] + pbuf[64 + j])
                    t = jnp.maximum(t, 0.0)
                    acc = acc + t * pbuf[128 + j]
                return acc
            acc = lax.fori_loop(0, 8, jblk, jnp.zeros((16,), jnp.float32))
            logit = acc + pbuf[192]
            logit = jnp.where(s16 == d16, jnp.float32(-1e9), logit)
            obuf[pl.ds(g * 16, 16)] = logit
        pltpu.sync_copy(obuf, out.at[pl.ds(base, CH)])
        return 0
    lax.fori_loop(0, NCHUNK, chunk, 0)


_sc_edge = pl.kernel(
    _edge_body,
    out_type=jax.ShapeDtypeStruct((E,), jnp.float32),
    mesh=_MESH,
    compiler_params=_SC_PARAMS,
    scratch_types=[
        pltpu.VMEM((CH,), jnp.int32),
        pltpu.VMEM((CH,), jnp.int32),
        pltpu.VMEM((CH,), jnp.float32),
        pltpu.VMEM((CH, 64), jnp.float32),
        pltpu.VMEM((CH, 64), jnp.float32),
        pltpu.VMEM((256,), jnp.float32),
        pltpu.SMEM((256,), jnp.float32),
        pltpu.VMEM((CH,), jnp.float32),
    ],
)


BR = 1000


def _tc1_body(x, p0, p1, w1s, w1n, b1, h1, deg):
    d = jnp.clip(p0[:, 128:129] + p1[:, 128:129], 1.0, None)
    agg = (p0[:, :128] + p1[:, :128]) / d
    h = x[...] @ w1s[...] + agg @ w1n[...] + b1[...]
    h1[...] = jnp.maximum(h, 0.0)
    deg[...] = d


def _tc1(x, p0, p1, w1s, w1n, b1):
    return pl.pallas_call(
        _tc1_body,
        grid=(N // BR,),
        in_specs=[
            pl.BlockSpec((BR, D), lambda i: (i, 0)),
            pl.BlockSpec((BR, WPAD), lambda i: (i, 0)),
            pl.BlockSpec((BR, WPAD), lambda i: (i, 0)),
            pl.BlockSpec((D, H), lambda i: (0, 0)),
            pl.BlockSpec((D, H), lambda i: (0, 0)),
            pl.BlockSpec((1, H), lambda i: (0, 0)),
        ],
        out_specs=[
            pl.BlockSpec((BR, H), lambda i: (i, 0)),
            pl.BlockSpec((BR, 1), lambda i: (i, 0)),
        ],
        out_shape=[
            jax.ShapeDtypeStruct((N, H), jnp.float32),
            jax.ShapeDtypeStruct((N, 1), jnp.float32),
        ],
    )(x, p0, p1, w1s, w1n, b1)


def _tc2_body(h1, q0, q1, deg, w2s, w2n, b2, we1a, we1b,
              wb1, bb1, wb2, bb2, wk1, bk1, wk2, bk2,
              psrc, pdst, bx, kc, ks):
    agg = (q0[...] + q1[...]) / deg[...]
    h2 = jnp.maximum(h1[...] @ w2s[...] + agg @ w2n[...] + b2[...], 0.0)
    psrc[...] = h2 @ we1a[...]
    pdst[...] = h2 @ we1b[...]
    tb = jnp.maximum(h2 @ wb1[...] + bb1[...], 0.0)
    bx[...] = tb @ wb2[...] + bb2[...]
    tk = jnp.maximum(h2 @ wk1[...] + bk1[...], 0.0)
    kl = tk @ wk2[...] + bk2[...]
    kcv = 1.0 + 7.0 * jax.nn.sigmoid(kl)
    kd = jnp.clip(jnp.round(kcv), 1.0, 8.0)
    kc[...] = kcv
    ks[...] = kcv + (kd - kcv)


def _tc2(h1, q0, q1, deg, pr):
    full = lambda a, b: pl.BlockSpec((a, b), lambda i: (0, 0))
    row = lambda b: pl.BlockSpec((BR, b), lambda i: (i, 0))
    return pl.pallas_call(
        _tc2_body,
        grid=(N // BR,),
        in_specs=[
            row(H), row(H), row(H), row(1),
            full(H, H), full(H, H), full(1, H),
            full(H, 64), full(H, 64),
            full(H, 64), full(1, 64), full(64, 2), full(1, 2),
            full(H, 32), full(1, 32), full(32, 1), full(1, 1),
        ],
        out_specs=[row(64), row(64), row(2), row(1), row(1)],
        out_shape=[
            jax.ShapeDtypeStruct((N, 64), jnp.float32),
            jax.ShapeDtypeStruct((N, 64), jnp.float32),
            jax.ShapeDtypeStruct((N, 2), jnp.float32),
            jax.ShapeDtypeStruct((N, 1), jnp.float32),
            jax.ShapeDtypeStruct((N, 1), jnp.float32),
        ],
    )(h1, q0, q1, deg, pr['W2s'], pr['W2n'], pr['b2'].reshape(1, H),
      pr['We1'][:H], pr['We1'][H:2 * H],
      pr['Wb1'], pr['bb1'].reshape(1, 64), pr['Wb2'], pr['bb2'].reshape(1, 2),
      pr['Wk1'], pr['bk1'].reshape(1, 32), pr['Wk2'], pr['bk2'].reshape(1, 1))


def kernel(x, edge_index, edge_weight, params):
    src = edge_index[0].astype(jnp.int32)
    dst = edge_index[1].astype(jnp.int32)
    xpad = jnp.concatenate(
        [x, jnp.ones((N, 1), jnp.float32), jnp.zeros((N, WPAD - D - 1), jnp.float32)],
        axis=1)

    p0, p1 = _sc_scatter_144(xpad, src, dst)
    h1, deg = _tc1(x, p0, p1, params['W1s'], params['W1n'],
                   params['b1'].reshape(1, H))
    q0, q1 = _sc_scatter_128(h1, src, dst)
    psrc, pdst, bx, kc, ks = _tc2(h1, q0, q1, deg, params)

    par = jnp.concatenate([
        params['We1'][2 * H],
        params['be1'],
        params['We2'][:, 0],
        jnp.broadcast_to(params['be2'], (64,)),
    ]).astype(jnp.float32)
    logits = _sc_edge(psrc, pdst, src, dst, edge_weight, par)

    return (logits, bx, kc[:, 0], ks[:, 0])


# speedup vs baseline: 4.8317x; 1.5182x over previous
import functools

import jax
import jax.numpy as jnp
from jax import lax
from jax.experimental import pallas as pl
from jax.experimental.pallas import tpu as pltpu
from jax.experimental.pallas import tpu_sc as plsc

N = 10000
E = 320000
D = 128
H = 128
WPAD = 144
NC = 2
NS = 16
NW = NC * NS
EW = E // NW
CH = 64
NCHUNK = EW // CH
TAIL = EW - NCHUNK * CH
CE = 256
NCE = -(-EW // CE)
RPT = N // NS
ZR = 125

_MESH = plsc.VectorSubcoreMesh(
    core_axis_name="c", subcore_axis_name="s", num_cores=NC, num_subcores=NS)
_SC_PARAMS = pltpu.CompilerParams(
    use_tc_tiling_on_sc=False, needs_layout_passes=False)


def _make_sc_scatter(W):
    def body(table, src, dst, out0, out1, srcb, dstb, gb0, gb1, acc,
             sg0, sg1, sa0, sa1):
        c = lax.axis_index("c")
        s = lax.axis_index("s")
        wid = c * NS + s

        def zrow(r, _):
            for cb in range(W // 16):
                gb0[r, pl.ds(cb * 16, 16)] = jnp.zeros((16,), jnp.float32)
            return 0
        lax.fori_loop(0, CH, zrow, 0)
        for kz in range(RPT // CH):
            pltpu.sync_copy(gb0, acc.at[pl.ds(s * RPT + kz * CH, CH)])
        pltpu.sync_copy(gb0.at[pl.ds(0, RPT % CH)],
                        acc.at[pl.ds(s * RPT + (RPT // CH) * CH, RPT % CH)])

        pltpu.sync_copy(src.at[pl.ds(wid * EW, EW)], srcb)
        pltpu.sync_copy(dst.at[pl.ds(wid * EW, EW)], dstb)
        plsc.subcore_barrier()

        bufs = ((gb0, sg0, sa0), (gb1, sg1, sa1))

        def gidx(k):
            return srcb.at[pl.ds(k * CH, CH)]

        def aidx(k):
            return dstb.at[pl.ds(k * CH, CH)]

        pltpu.async_copy(table.at[gidx(0)], gb0, sg0)

        def pair(p, _):
            for b in range(2):
                k = p * 2 + b
                buf, sg, sa = bufs[b]
                obuf, osg, osa = bufs[1 - b]
                pltpu.make_async_copy(table.at[gidx(k)], buf, sg).wait()
                @pl.when(k >= 1)
                def _():
                    pltpu.make_async_copy(obuf, acc.at[aidx(k - 1)], osa).wait()
                @pl.when(k + 1 < NCHUNK)
                def _():
                    pltpu.async_copy(table.at[gidx(k + 1)], obuf, osg)
                pltpu.async_copy(buf, acc.at[aidx(k)], sa, add=True)
            return 0
        lax.fori_loop(0, NCHUNK // 2, pair, 0)
        tb = NCHUNK * CH
        pltpu.sync_copy(table.at[srcb.at[pl.ds(tb, TAIL)]],
                        gb0.at[pl.ds(0, TAIL)])
        pltpu.sync_copy(gb0.at[pl.ds(0, TAIL)],
                        acc.at[dstb.at[pl.ds(tb, TAIL)]], add=True)
        pltpu.make_async_copy(gb1, acc.at[aidx(NCHUNK - 1)], sa1).wait()
        plsc.subcore_barrier()

        rows = pl.ds(s * RPT, RPT)

        @pl.when(c == 0)
        def _():
            pltpu.sync_copy(acc.at[rows], out0.at[rows])

        @pl.when(c == 1)
        def _():
            pltpu.sync_copy(acc.at[rows], out1.at[rows])

    sds = jax.ShapeDtypeStruct((N, W), jnp.float32)
    return pl.kernel(
        body,
        out_type=(sds, sds),
        mesh=_MESH,
        compiler_params=_SC_PARAMS,
        scratch_types=[
            pltpu.VMEM((EW,), jnp.int32),
            pltpu.VMEM((EW,), jnp.int32),
            pltpu.VMEM((CH, W), jnp.float32),
            pltpu.VMEM((CH, W), jnp.float32),
            pltpu.VMEM_SHARED((N, W), jnp.float32),
            pltpu.SemaphoreType.DMA,
            pltpu.SemaphoreType.DMA,
            pltpu.SemaphoreType.DMA,
            pltpu.SemaphoreType.DMA,
        ],
    )


_sc_scatter_144 = _make_sc_scatter(WPAD)
_sc_scatter_128 = _make_sc_scatter(H)


def _edge_body(psrc, pdst, src, dst, ew, par, out,
               srcb, dstb, wb, bS0, bS1, bD0, bD1, pvbuf, pbuf, ob0, ob1,
               ss0, ss1, sd0, sd1, so0, so1):
    c = lax.axis_index("c")
    s = lax.axis_index("s")
    wid = c * NS + s
    pltpu.sync_copy(par, pvbuf)
    for i in range(16):
        v = pvbuf[pl.ds(i * 16, 16)]
        for l in range(16):
            pbuf[i * 16 + l] = v[l]
    ebase = wid * EW
    pltpu.sync_copy(src.at[pl.ds(ebase, EW)], srcb)
    pltpu.sync_copy(dst.at[pl.ds(ebase, EW)], dstb)
    pltpu.sync_copy(ew.at[pl.ds(ebase, EW)], wb)
    lanes = lax.iota(jnp.int32, 16)

    def off(k):
        return jnp.minimum(k * CE, EW - CE)

    def sidx(k):
        return srcb.at[pl.ds(off(k), CE)]

    def didx(k):
        return dstb.at[pl.ds(off(k), CE)]

    bufs = ((bS0, bD0, ob0, ss0, sd0, so0), (bS1, bD1, ob1, ss1, sd1, so1))
    pltpu.async_copy(psrc.at[sidx(0)], bS0, ss0)
    pltpu.async_copy(pdst.at[didx(0)], bD0, sd0)

    def pair(p, _):
        for b in range(2):
            k = p * 2 + b
            bS, bD, ob, ss, sd, so = bufs[b]
            oS, oD, oob, oss, osd, oso = bufs[1 - b]
            pltpu.make_async_copy(psrc.at[sidx(k)], bS, ss).wait()
            pltpu.make_async_copy(pdst.at[didx(k)], bD, sd).wait()

            @pl.when(k + 1 < NCE)
            def _():
                pltpu.async_copy(psrc.at[sidx(k + 1)], oS, oss)
                pltpu.async_copy(pdst.at[didx(k + 1)], oD, osd)

            @pl.when(k >= 2)
            def _():
                pltpu.make_async_copy(ob, out.at[pl.ds(off(k - 2) + ebase, CE)],
                                      so).wait()
            ko = off(k)
            for g in range(CE // 16):
                rows = lanes + g * 16
                w16 = wb[pl.ds(ko + g * 16, 16)]
                s16 = srcb[pl.ds(ko + g * 16, 16)]
                d16 = dstb[pl.ds(ko + g * 16, 16)]

                def jblk(jo, acc):
                    for ju in range(8):
                        j = jo * 8 + ju
                        jb = jnp.full((16,), j, jnp.int32)
                        gs = plsc.load_gather(bS, [rows, jb])
                        gd = plsc.load_gather(bD, [rows, jb])
                        t = gs + gd + (w16 * pbuf[j] + pbuf[64 + j])
                        t = jnp.maximum(t, 0.0)
                        acc = acc + t * pbuf[128 + j]
                    return acc
                acc = lax.fori_loop(0, 8, jblk, jnp.zeros((16,), jnp.float32))
                logit = acc + pbuf[192]
                logit = jnp.where(s16 == d16, jnp.float32(-1e9), logit)
                ob[pl.ds(g * 16, 16)] = logit
            pltpu.async_copy(ob, out.at[pl.ds(ko + ebase, CE)], so)
        return 0
    lax.fori_loop(0, NCE // 2, pair, 0)
    pltpu.make_async_copy(ob0, out.at[pl.ds(off(NCE - 2) + ebase, CE)], so0).wait()
    pltpu.make_async_copy(ob1, out.at[pl.ds(off(NCE - 1) + ebase, CE)], so1).wait()


_sc_edge = pl.kernel(
    _edge_body,
    out_type=jax.ShapeDtypeStruct((E,), jnp.float32),
    mesh=_MESH,
    compiler_params=_SC_PARAMS,
    scratch_types=[
        pltpu.VMEM((EW,), jnp.int32),
        pltpu.VMEM((EW,), jnp.int32),
        pltpu.VMEM((EW,), jnp.float32),
        pltpu.VMEM((CE, 64), jnp.float32),
        pltpu.VMEM((CE, 64), jnp.float32),
        pltpu.VMEM((CE, 64), jnp.float32),
        pltpu.VMEM((CE, 64), jnp.float32),
        pltpu.VMEM((256,), jnp.float32),
        pltpu.SMEM((256,), jnp.float32),
        pltpu.VMEM((CE,), jnp.float32),
        pltpu.VMEM((CE,), jnp.float32),
        pltpu.SemaphoreType.DMA,
        pltpu.SemaphoreType.DMA,
        pltpu.SemaphoreType.DMA,
        pltpu.SemaphoreType.DMA,
        pltpu.SemaphoreType.DMA,
        pltpu.SemaphoreType.DMA,
    ],
)


BR = 1000


def _tc1_body(x, p0, p1, w1s, w1n, b1, h1, deg):
    d = jnp.clip(p0[:, 128:129] + p1[:, 128:129], 1.0, None)
    agg = (p0[:, :128] + p1[:, :128]) / d
    h = x[...] @ w1s[...] + agg @ w1n[...] + b1[...]
    h1[...] = jnp.maximum(h, 0.0)
    deg[...] = d


def _tc1(x, p0, p1, w1s, w1n, b1):
    return pl.pallas_call(
        _tc1_body,
        grid=(N // BR,),
        in_specs=[
            pl.BlockSpec((BR, D), lambda i: (i, 0)),
            pl.BlockSpec((BR, WPAD), lambda i: (i, 0)),
            pl.BlockSpec((BR, WPAD), lambda i: (i, 0)),
            pl.BlockSpec((D, H), lambda i: (0, 0)),
            pl.BlockSpec((D, H), lambda i: (0, 0)),
            pl.BlockSpec((1, H), lambda i: (0, 0)),
        ],
        out_specs=[
            pl.BlockSpec((BR, H), lambda i: (i, 0)),
            pl.BlockSpec((BR, 1), lambda i: (i, 0)),
        ],
        out_shape=[
            jax.ShapeDtypeStruct((N, H), jnp.float32),
            jax.ShapeDtypeStruct((N, 1), jnp.float32),
        ],
    )(x, p0, p1, w1s, w1n, b1)


def _tc2_body(h1, q0, q1, deg, w2s, w2n, b2, we1a, we1b,
              wb1, bb1, wb2, bb2, wk1, bk1, wk2, bk2,
              psrc, pdst, bx, kc, ks):
    agg = (q0[...] + q1[...]) / deg[...]
    h2 = jnp.maximum(h1[...] @ w2s[...] + agg @ w2n[...] + b2[...], 0.0)
    psrc[...] = h2 @ we1a[...]
    pdst[...] = h2 @ we1b[...]
    tb = jnp.maximum(h2 @ wb1[...] + bb1[...], 0.0)
    bx[...] = tb @ wb2[...] + bb2[...]
    tk = jnp.maximum(h2 @ wk1[...] + bk1[...], 0.0)
    kl = tk @ wk2[...] + bk2[...]
    kcv = 1.0 + 7.0 * jax.nn.sigmoid(kl)
    kd = jnp.clip(jnp.round(kcv), 1.0, 8.0)
    kc[...] = kcv
    ks[...] = kcv + (kd - kcv)


def _tc2(h1, q0, q1, deg, pr):
    full = lambda a, b: pl.BlockSpec((a, b), lambda i: (0, 0))
    row = lambda b: pl.BlockSpec((BR, b), lambda i: (i, 0))
    return pl.pallas_call(
        _tc2_body,
        grid=(N // BR,),
        in_specs=[
            row(H), row(H), row(H), row(1),
            full(H, H), full(H, H), full(1, H),
            full(H, 64), full(H, 64),
            full(H, 64), full(1, 64), full(64, 2), full(1, 2),
            full(H, 32), full(1, 32), full(32, 1), full(1, 1),
        ],
        out_specs=[row(64), row(64), row(2), row(1), row(1)],
        out_shape=[
            jax.ShapeDtypeStruct((N, 64), jnp.float32),
            jax.ShapeDtypeStruct((N, 64), jnp.float32),
            jax.ShapeDtypeStruct((N, 2), jnp.float32),
            jax.ShapeDtypeStruct((N, 1), jnp.float32),
            jax.ShapeDtypeStruct((N, 1), jnp.float32),
        ],
    )(h1, q0, q1, deg, pr['W2s'], pr['W2n'], pr['b2'].reshape(1, H),
      pr['We1'][:H], pr['We1'][H:2 * H],
      pr['Wb1'], pr['bb1'].reshape(1, 64), pr['Wb2'], pr['bb2'].reshape(1, 2),
      pr['Wk1'], pr['bk1'].reshape(1, 32), pr['Wk2'], pr['bk2'].reshape(1, 1))


def kernel(x, edge_index, edge_weight, params):
    src = edge_index[0].astype(jnp.int32)
    dst = edge_index[1].astype(jnp.int32)
    xpad = jnp.concatenate(
        [x, jnp.ones((N, 1), jnp.float32), jnp.zeros((N, WPAD - D - 1), jnp.float32)],
        axis=1)

    p0, p1 = _sc_scatter_144(xpad, src, dst)
    h1, deg = _tc1(x, p0, p1, params['W1s'], params['W1n'],
                   params['b1'].reshape(1, H))
    q0, q1 = _sc_scatter_128(h1, src, dst)
    psrc, pdst, bx, kc, ks = _tc2(h1, q0, q1, deg, params)

    par = jnp.concatenate([
        params['We1'][2 * H],
        params['be1'],
        params['We2'][:, 0],
        jnp.broadcast_to(params['be2'], (64,)),
    ]).astype(jnp.float32)
    logits = _sc_edge(psrc, pdst, src, dst, edge_weight, par)

    return (logits, bx, kc[:, 0], ks[:, 0])


# speedup vs baseline: 5.1547x; 1.0669x over previous
import functools

import jax
import jax.numpy as jnp
from jax import lax
from jax.experimental import pallas as pl
from jax.experimental.pallas import tpu as pltpu
from jax.experimental.pallas import tpu_sc as plsc

N = 10000
E = 320000
D = 128
H = 128
WPAD = 144
NC = 2
NS = 16
NW = NC * NS
EW = E // NW
CH = 64
NCHUNK = EW // CH
TAIL = EW - NCHUNK * CH
CE = 256
NCE = -(-EW // CE)
RPT = N // NS
ZR = 125

_MESH = plsc.VectorSubcoreMesh(
    core_axis_name="c", subcore_axis_name="s", num_cores=NC, num_subcores=NS)
_SC_PARAMS = pltpu.CompilerParams(
    use_tc_tiling_on_sc=False, needs_layout_passes=False)


def _make_sc_scatter(W):
    def body(table, src, dst, out0, out1, srcb, dstb, gb0, gb1, acc,
             sg0, sg1, sa0, sa1):
        c = lax.axis_index("c")
        s = lax.axis_index("s")
        wid = c * NS + s

        def zrow(r, _):
            for cb in range(W // 16):
                gb0[r, pl.ds(cb * 16, 16)] = jnp.zeros((16,), jnp.float32)
            return 0
        lax.fori_loop(0, CH, zrow, 0)
        for kz in range(RPT // CH):
            pltpu.sync_copy(gb0, acc.at[pl.ds(s * RPT + kz * CH, CH)])
        pltpu.sync_copy(gb0.at[pl.ds(0, RPT % CH)],
                        acc.at[pl.ds(s * RPT + (RPT // CH) * CH, RPT % CH)])

        pltpu.sync_copy(src.at[pl.ds(wid * EW, EW)], srcb)
        pltpu.sync_copy(dst.at[pl.ds(wid * EW, EW)], dstb)
        plsc.subcore_barrier()

        bufs = ((gb0, sg0, sa0), (gb1, sg1, sa1))

        def gidx(k):
            return srcb.at[pl.ds(k * CH, CH)]

        def aidx(k):
            return dstb.at[pl.ds(k * CH, CH)]

        pltpu.async_copy(table.at[gidx(0)], gb0, sg0)

        def pair(p, _):
            for b in range(2):
                k = p * 2 + b
                buf, sg, sa = bufs[b]
                obuf, osg, osa = bufs[1 - b]
                pltpu.make_async_copy(table.at[gidx(k)], buf, sg).wait()
                @pl.when(k >= 1)
                def _():
                    pltpu.make_async_copy(obuf, acc.at[aidx(k - 1)], osa).wait()
                @pl.when(k + 1 < NCHUNK)
                def _():
                    pltpu.async_copy(table.at[gidx(k + 1)], obuf, osg)
                pltpu.async_copy(buf, acc.at[aidx(k)], sa, add=True)
            return 0
        lax.fori_loop(0, NCHUNK // 2, pair, 0)
        tb = NCHUNK * CH
        pltpu.sync_copy(table.at[srcb.at[pl.ds(tb, TAIL)]],
                        gb0.at[pl.ds(0, TAIL)])
        pltpu.sync_copy(gb0.at[pl.ds(0, TAIL)],
                        acc.at[dstb.at[pl.ds(tb, TAIL)]], add=True)
        pltpu.make_async_copy(gb1, acc.at[aidx(NCHUNK - 1)], sa1).wait()
        plsc.subcore_barrier()

        rows = pl.ds(s * RPT, RPT)

        @pl.when(c == 0)
        def _():
            pltpu.sync_copy(acc.at[rows], out0.at[rows])

        @pl.when(c == 1)
        def _():
            pltpu.sync_copy(acc.at[rows], out1.at[rows])

    sds = jax.ShapeDtypeStruct((N, W), jnp.float32)
    return pl.kernel(
        body,
        out_type=(sds, sds),
        mesh=_MESH,
        compiler_params=_SC_PARAMS,
        scratch_types=[
            pltpu.VMEM((EW,), jnp.int32),
            pltpu.VMEM((EW,), jnp.int32),
            pltpu.VMEM((CH, W), jnp.float32),
            pltpu.VMEM((CH, W), jnp.float32),
            pltpu.VMEM_SHARED((N, W), jnp.float32),
            pltpu.SemaphoreType.DMA,
            pltpu.SemaphoreType.DMA,
            pltpu.SemaphoreType.DMA,
            pltpu.SemaphoreType.DMA,
        ],
    )


_sc_scatter_144 = _make_sc_scatter(WPAD)
_sc_scatter_128 = _make_sc_scatter(H)


def _edge_body(psrc, pdst, src, dst, ew, par, out,
               srcb, dstb, wb, bS0, bS1, bD0, bD1, pvbuf, pbuf, ob0, ob1,
               ss0, ss1, sd0, sd1, so0, so1):
    c = lax.axis_index("c")
    s = lax.axis_index("s")
    wid = c * NS + s
    pltpu.sync_copy(par, pvbuf)
    for i in range(16):
        v = pvbuf[pl.ds(i * 16, 16)]
        for l in range(16):
            pbuf[i * 16 + l] = v[l]
    ebase = wid * EW
    pltpu.sync_copy(src.at[pl.ds(ebase, EW)], srcb)
    pltpu.sync_copy(dst.at[pl.ds(ebase, EW)], dstb)
    pltpu.sync_copy(ew.at[pl.ds(ebase, EW)], wb)
    lanes = lax.iota(jnp.int32, 16)

    def off(k):
        return jnp.minimum(k * CE, EW - CE)

    def sidx(k):
        return srcb.at[pl.ds(off(k), CE)]

    def didx(k):
        return dstb.at[pl.ds(off(k), CE)]

    bufs = ((bS0, bD0, ob0, ss0, sd0, so0), (bS1, bD1, ob1, ss1, sd1, so1))
    pltpu.async_copy(psrc.at[sidx(0)], bS0, ss0)
    pltpu.async_copy(pdst.at[didx(0)], bD0, sd0)

    def pair(p, _):
        for b in range(2):
            k = p * 2 + b
            bS, bD, ob, ss, sd, so = bufs[b]
            oS, oD, oob, oss, osd, oso = bufs[1 - b]
            pltpu.make_async_copy(psrc.at[sidx(k)], bS, ss).wait()
            pltpu.make_async_copy(pdst.at[didx(k)], bD, sd).wait()

            @pl.when(k + 1 < NCE)
            def _():
                pltpu.async_copy(psrc.at[sidx(k + 1)], oS, oss)
                pltpu.async_copy(pdst.at[didx(k + 1)], oD, osd)

            @pl.when(k >= 2)
            def _():
                pltpu.make_async_copy(ob, out.at[pl.ds(off(k - 2) + ebase, CE)],
                                      so).wait()
            ko = off(k)
            for gp in range(CE // 32):
                gA, gB = 2 * gp, 2 * gp + 1
                rA = lanes + gA * 16
                rB = lanes + gB * 16
                wA = wb[pl.ds(ko + gA * 16, 16)]
                wB = wb[pl.ds(ko + gB * 16, 16)]

                def jblk(jo, accs):
                    a0, a1, b0, b1 = accs
                    aa = [a0, a1]
                    bb = [b0, b1]
                    for ju in range(8):
                        j = jo * 8 + ju
                        jb = jnp.full((16,), j, jnp.int32)
                        av = jnp.full((16,), pbuf[j], jnp.float32)
                        bv = jnp.full((16,), pbuf[64 + j], jnp.float32)
                        cv = jnp.full((16,), pbuf[128 + j], jnp.float32)
                        gsA = plsc.load_gather(bS, [rA, jb])
                        gdA = plsc.load_gather(bD, [rA, jb])
                        gsB = plsc.load_gather(bS, [rB, jb])
                        gdB = plsc.load_gather(bD, [rB, jb])
                        tA = jnp.maximum((gsA + gdA) + (wA * av + bv), 0.0)
                        tB = jnp.maximum((gsB + gdB) + (wB * av + bv), 0.0)
                        aa[ju % 2] = aa[ju % 2] + tA * cv
                        bb[ju % 2] = bb[ju % 2] + tB * cv
                    return aa[0], aa[1], bb[0], bb[1]
                z16 = jnp.zeros((16,), jnp.float32)
                a0, a1, b0, b1 = lax.fori_loop(0, 8, jblk, (z16, z16, z16, z16))
                for g, acc in ((gA, a0 + a1), (gB, b0 + b1)):
                    s16 = srcb[pl.ds(ko + g * 16, 16)]
                    d16 = dstb[pl.ds(ko + g * 16, 16)]
                    logit = acc + pbuf[192]
                    logit = jnp.where(s16 == d16, jnp.float32(-1e9), logit)
                    ob[pl.ds(g * 16, 16)] = logit
            pltpu.async_copy(ob, out.at[pl.ds(ko + ebase, CE)], so)
        return 0
    lax.fori_loop(0, NCE // 2, pair, 0)
    pltpu.make_async_copy(ob0, out.at[pl.ds(off(NCE - 2) + ebase, CE)], so0).wait()
    pltpu.make_async_copy(ob1, out.at[pl.ds(off(NCE - 1) + ebase, CE)], so1).wait()


_sc_edge = pl.kernel(
    _edge_body,
    out_type=jax.ShapeDtypeStruct((E,), jnp.float32),
    mesh=_MESH,
    compiler_params=_SC_PARAMS,
    scratch_types=[
        pltpu.VMEM((EW,), jnp.int32),
        pltpu.VMEM((EW,), jnp.int32),
        pltpu.VMEM((EW,), jnp.float32),
        pltpu.VMEM((CE, 64), jnp.float32),
        pltpu.VMEM((CE, 64), jnp.float32),
        pltpu.VMEM((CE, 64), jnp.float32),
        pltpu.VMEM((CE, 64), jnp.float32),
        pltpu.VMEM((256,), jnp.float32),
        pltpu.SMEM((256,), jnp.float32),
        pltpu.VMEM((CE,), jnp.float32),
        pltpu.VMEM((CE,), jnp.float32),
        pltpu.SemaphoreType.DMA,
        pltpu.SemaphoreType.DMA,
        pltpu.SemaphoreType.DMA,
        pltpu.SemaphoreType.DMA,
        pltpu.SemaphoreType.DMA,
        pltpu.SemaphoreType.DMA,
    ],
)


BR = 1000


def _tc1_body(x, p0, p1, w1s, w1n, b1, h1, deg):
    d = jnp.clip(p0[:, 128:129] + p1[:, 128:129], 1.0, None)
    agg = (p0[:, :128] + p1[:, :128]) / d
    h = x[...] @ w1s[...] + agg @ w1n[...] + b1[...]
    h1[...] = jnp.maximum(h, 0.0)
    deg[...] = d


def _tc1(x, p0, p1, w1s, w1n, b1):
    return pl.pallas_call(
        _tc1_body,
        grid=(N // BR,),
        in_specs=[
            pl.BlockSpec((BR, D), lambda i: (i, 0)),
            pl.BlockSpec((BR, WPAD), lambda i: (i, 0)),
            pl.BlockSpec((BR, WPAD), lambda i: (i, 0)),
            pl.BlockSpec((D, H), lambda i: (0, 0)),
            pl.BlockSpec((D, H), lambda i: (0, 0)),
            pl.BlockSpec((1, H), lambda i: (0, 0)),
        ],
        out_specs=[
            pl.BlockSpec((BR, H), lambda i: (i, 0)),
            pl.BlockSpec((BR, 1), lambda i: (i, 0)),
        ],
        out_shape=[
            jax.ShapeDtypeStruct((N, H), jnp.float32),
            jax.ShapeDtypeStruct((N, 1), jnp.float32),
        ],
    )(x, p0, p1, w1s, w1n, b1)


def _tc2_body(h1, q0, q1, deg, w2s, w2n, b2, we1a, we1b,
              wb1, bb1, wb2, bb2, wk1, bk1, wk2, bk2,
              psrc, pdst, bx, kc, ks):
    agg = (q0[...] + q1[...]) / deg[...]
    h2 = jnp.maximum(h1[...] @ w2s[...] + agg @ w2n[...] + b2[...], 0.0)
    psrc[...] = h2 @ we1a[...]
    pdst[...] = h2 @ we1b[...]
    tb = jnp.maximum(h2 @ wb1[...] + bb1[...], 0.0)
    bx[...] = tb @ wb2[...] + bb2[...]
    tk = jnp.maximum(h2 @ wk1[...] + bk1[...], 0.0)
    kl = tk @ wk2[...] + bk2[...]
    kcv = 1.0 + 7.0 * jax.nn.sigmoid(kl)
    kd = jnp.clip(jnp.round(kcv), 1.0, 8.0)
    kc[...] = kcv
    ks[...] = kcv + (kd - kcv)


def _tc2(h1, q0, q1, deg, pr):
    full = lambda a, b: pl.BlockSpec((a, b), lambda i: (0, 0))
    row = lambda b: pl.BlockSpec((BR, b), lambda i: (i, 0))
    return pl.pallas_call(
        _tc2_body,
        grid=(N // BR,),
        in_specs=[
            row(H), row(H), row(H), row(1),
            full(H, H), full(H, H), full(1, H),
            full(H, 64), full(H, 64),
            full(H, 64), full(1, 64), full(64, 2), full(1, 2),
            full(H, 32), full(1, 32), full(32, 1), full(1, 1),
        ],
        out_specs=[row(64), row(64), row(2), row(1), row(1)],
        out_shape=[
            jax.ShapeDtypeStruct((N, 64), jnp.float32),
            jax.ShapeDtypeStruct((N, 64), jnp.float32),
            jax.ShapeDtypeStruct((N, 2), jnp.float32),
            jax.ShapeDtypeStruct((N, 1), jnp.float32),
            jax.ShapeDtypeStruct((N, 1), jnp.float32),
        ],
    )(h1, q0, q1, deg, pr['W2s'], pr['W2n'], pr['b2'].reshape(1, H),
      pr['We1'][:H], pr['We1'][H:2 * H],
      pr['Wb1'], pr['bb1'].reshape(1, 64), pr['Wb2'], pr['bb2'].reshape(1, 2),
      pr['Wk1'], pr['bk1'].reshape(1, 32), pr['Wk2'], pr['bk2'].reshape(1, 1))


def kernel(x, edge_index, edge_weight, params):
    src = edge_index[0].astype(jnp.int32)
    dst = edge_index[1].astype(jnp.int32)
    xpad = jnp.concatenate(
        [x, jnp.ones((N, 1), jnp.float32), jnp.zeros((N, WPAD - D - 1), jnp.float32)],
        axis=1)

    p0, p1 = _sc_scatter_144(xpad, src, dst)
    h1, deg = _tc1(x, p0, p1, params['W1s'], params['W1n'],
                   params['b1'].reshape(1, H))
    q0, q1 = _sc_scatter_128(h1, src, dst)
    psrc, pdst, bx, kc, ks = _tc2(h1, q0, q1, deg, params)

    par = jnp.concatenate([
        params['We1'][2 * H],
        params['be1'],
        params['We2'][:, 0],
        jnp.broadcast_to(params['be2'], (64,)),
    ]).astype(jnp.float32)
    logits = _sc_edge(psrc, pdst, src, dst, edge_weight, par)

    return (logits, bx, kc[:, 0], ks[:, 0])


# speedup vs baseline: 9.9971x; 1.9394x over previous
import functools

import jax
import jax.numpy as jnp
from jax import lax
from jax.experimental import pallas as pl
from jax.experimental.pallas import tpu as pltpu
from jax.experimental.pallas import tpu_sc as plsc

N = 10000
E = 320000
D = 128
H = 128
WPAD = 144
NC = 2
NS = 16
NW = NC * NS
EW = E // NW
CH = 64
NCHUNK = EW // CH
TAIL = EW - NCHUNK * CH
CE = 256
NCE = -(-EW // CE)
RPT = N // NS
ZR = 125

_TAKE_DNUMS = lax.GatherDimensionNumbers(
    offset_dims=(), collapsed_slice_dims=(0,), start_index_map=(0,))


def _take16(x, idx):
    return lax.gather(x, idx[:, None], _TAKE_DNUMS, slice_sizes=(1,),
                      mode=lax.GatherScatterMode.PROMISE_IN_BOUNDS)


_MESH = plsc.VectorSubcoreMesh(
    core_axis_name="c", subcore_axis_name="s", num_cores=NC, num_subcores=NS)
_SC_PARAMS = pltpu.CompilerParams(
    use_tc_tiling_on_sc=False, needs_layout_passes=False)


def _make_sc_scatter(W):
    def body(table, src, dst, out0, out1, srcb, dstb, gb0, gb1, acc,
             sg0, sg1, sa0, sa1):
        c = lax.axis_index("c")
        s = lax.axis_index("s")
        wid = c * NS + s

        def zrow(r, _):
            for cb in range(W // 16):
                gb0[r, pl.ds(cb * 16, 16)] = jnp.zeros((16,), jnp.float32)
            return 0
        lax.fori_loop(0, CH, zrow, 0)
        for kz in range(RPT // CH):
            pltpu.sync_copy(gb0, acc.at[pl.ds(s * RPT + kz * CH, CH)])
        pltpu.sync_copy(gb0.at[pl.ds(0, RPT % CH)],
                        acc.at[pl.ds(s * RPT + (RPT // CH) * CH, RPT % CH)])

        pltpu.sync_copy(src.at[pl.ds(wid * EW, EW)], srcb)
        pltpu.sync_copy(dst.at[pl.ds(wid * EW, EW)], dstb)
        plsc.subcore_barrier()

        bufs = ((gb0, sg0, sa0), (gb1, sg1, sa1))

        def gidx(k):
            return srcb.at[pl.ds(k * CH, CH)]

        def aidx(k):
            return dstb.at[pl.ds(k * CH, CH)]

        pltpu.async_copy(table.at[gidx(0)], gb0, sg0)

        def pair(p, _):
            for b in range(2):
                k = p * 2 + b
                buf, sg, sa = bufs[b]
                obuf, osg, osa = bufs[1 - b]
                pltpu.make_async_copy(table.at[gidx(k)], buf, sg).wait()
                @pl.when(k >= 1)
                def _():
                    pltpu.make_async_copy(obuf, acc.at[aidx(k - 1)], osa).wait()
                @pl.when(k + 1 < NCHUNK)
                def _():
                    pltpu.async_copy(table.at[gidx(k + 1)], obuf, osg)
                pltpu.async_copy(buf, acc.at[aidx(k)], sa, add=True)
            return 0
        lax.fori_loop(0, NCHUNK // 2, pair, 0)
        tb = NCHUNK * CH
        pltpu.sync_copy(table.at[srcb.at[pl.ds(tb, TAIL)]],
                        gb0.at[pl.ds(0, TAIL)])
        pltpu.sync_copy(gb0.at[pl.ds(0, TAIL)],
                        acc.at[dstb.at[pl.ds(tb, TAIL)]], add=True)
        pltpu.make_async_copy(gb1, acc.at[aidx(NCHUNK - 1)], sa1).wait()
        plsc.subcore_barrier()

        rows = pl.ds(s * RPT, RPT)

        @pl.when(c == 0)
        def _():
            pltpu.sync_copy(acc.at[rows], out0.at[rows])

        @pl.when(c == 1)
        def _():
            pltpu.sync_copy(acc.at[rows], out1.at[rows])

    sds = jax.ShapeDtypeStruct((N, W), jnp.float32)
    return pl.kernel(
        body,
        out_type=(sds, sds),
        mesh=_MESH,
        compiler_params=_SC_PARAMS,
        scratch_types=[
            pltpu.VMEM((EW,), jnp.int32),
            pltpu.VMEM((EW,), jnp.int32),
            pltpu.VMEM((CH, W), jnp.float32),
            pltpu.VMEM((CH, W), jnp.float32),
            pltpu.VMEM_SHARED((N, W), jnp.float32),
            pltpu.SemaphoreType.DMA,
            pltpu.SemaphoreType.DMA,
            pltpu.SemaphoreType.DMA,
            pltpu.SemaphoreType.DMA,
        ],
    )


_sc_scatter_144 = _make_sc_scatter(WPAD)
_sc_scatter_128 = _make_sc_scatter(H)


def _edge_body(psrc, pdst, src, dst, ew, par, out,
               srcb, dstb, wb, bS0, bS1, bD0, bD1,
               pvbuf, ob0, ob1,
               ss0, ss1, sd0, sd1, so0, so1):
    c = lax.axis_index("c")
    s = lax.axis_index("s")
    wid = c * NS + s
    pltpu.sync_copy(par, pvbuf)
    ebase = wid * EW
    pltpu.sync_copy(src.at[pl.ds(ebase, EW)], srcb)
    pltpu.sync_copy(dst.at[pl.ds(ebase, EW)], dstb)
    pltpu.sync_copy(ew.at[pl.ds(ebase, EW)], wb)
    lanes = lax.iota(jnp.int32, 16)

    def off(k):
        return jnp.minimum(k * CE, EW - CE)

    def sidx(k):
        return srcb.at[pl.ds(off(k), CE)]

    def didx(k):
        return dstb.at[pl.ds(off(k), CE)]

    av = [pvbuf[pl.ds(c * 16, 16)] for c in range(4)]
    bv = [pvbuf[pl.ds(64 + c * 16, 16)] for c in range(4)]
    cv = [pvbuf[pl.ds(128 + c * 16, 16)] for c in range(4)]
    bev = pvbuf[pl.ds(192, 16)]
    perms = [lanes ^ dd for dd in (8, 4, 2, 1)]

    bufs = ((bS0, bD0, ob0, ss0, sd0, so0), (bS1, bD1, ob1, ss1, sd1, so1))
    pltpu.async_copy(psrc.at[sidx(0)], bS0, ss0)
    pltpu.async_copy(pdst.at[didx(0)], bD0, sd0)

    def pair(p, _):
        for b in range(2):
            k = p * 2 + b
            bS, bD, ob, ss, sd, so = bufs[b]
            oS, oD, oob, oss, osd, oso = bufs[1 - b]
            pltpu.make_async_copy(psrc.at[sidx(k)], bS, ss).wait()
            pltpu.make_async_copy(pdst.at[didx(k)], bD, sd).wait()

            @pl.when(k + 1 < NCE)
            def _():
                pltpu.async_copy(psrc.at[sidx(k + 1)], oS, oss)
                pltpu.async_copy(pdst.at[didx(k + 1)], oD, osd)

            @pl.when(k >= 2)
            def _():
                pltpu.make_async_copy(ob, out.at[pl.ds(off(k - 2) + ebase, CE)],
                                      so).wait()
            ko = off(k)

            def grp(g, _):
                gb = g * 16
                w16 = wb[pl.ds(ko + gb, 16)]
                s16 = srcb[pl.ds(ko + gb, 16)]
                d16 = dstb[pl.ds(ko + gb, 16)]
                res = bev
                for el in range(16):
                    e = gb + el
                    wv = jnp.full((16,), w16[el], jnp.float32)
                    p16 = None
                    for cc in range(4):
                        gs = bS[e, pl.ds(cc * 16, 16)]
                        gd = bD[e, pl.ds(cc * 16, 16)]
                        t = jnp.maximum((gs + gd) + (wv * av[cc] + bv[cc]), 0.0)
                        tc = t * cv[cc]
                        p16 = tc if p16 is None else p16 + tc
                    for pm in perms:
                        p16 = p16 + _take16(p16, pm)
                    res = jnp.where(lanes == el, p16, res)
                logit = jnp.where(s16 == d16, jnp.float32(-1e9), res)
                ob[pl.ds(gb, 16)] = logit
                return 0
            lax.fori_loop(0, CE // 16, grp, 0)
            pltpu.async_copy(ob, out.at[pl.ds(ko + ebase, CE)], so)
        return 0
    lax.fori_loop(0, NCE // 2, pair, 0)
    pltpu.make_async_copy(ob0, out.at[pl.ds(off(NCE - 2) + ebase, CE)], so0).wait()
    pltpu.make_async_copy(ob1, out.at[pl.ds(off(NCE - 1) + ebase, CE)], so1).wait()


_sc_edge = pl.kernel(
    _edge_body,
    out_type=jax.ShapeDtypeStruct((E,), jnp.float32),
    mesh=_MESH,
    compiler_params=_SC_PARAMS,
    scratch_types=[
        pltpu.VMEM((EW,), jnp.int32),
        pltpu.VMEM((EW,), jnp.int32),
        pltpu.VMEM((EW,), jnp.float32),
        pltpu.VMEM((CE, 64), jnp.float32),
        pltpu.VMEM((CE, 64), jnp.float32),
        pltpu.VMEM((CE, 64), jnp.float32),
        pltpu.VMEM((CE, 64), jnp.float32),
        pltpu.VMEM((256,), jnp.float32),
        pltpu.VMEM((CE,), jnp.float32),
        pltpu.VMEM((CE,), jnp.float32),
        pltpu.SemaphoreType.DMA,
        pltpu.SemaphoreType.DMA,
        pltpu.SemaphoreType.DMA,
        pltpu.SemaphoreType.DMA,
        pltpu.SemaphoreType.DMA,
        pltpu.SemaphoreType.DMA,
    ],
)


BR = 1000


def _tc1_body(x, p0, p1, w1s, w1n, b1, h1, deg):
    d = jnp.clip(p0[:, 128:129] + p1[:, 128:129], 1.0, None)
    agg = (p0[:, :128] + p1[:, :128]) / d
    h = x[...] @ w1s[...] + agg @ w1n[...] + b1[...]
    h1[...] = jnp.maximum(h, 0.0)
    deg[...] = d


def _tc1(x, p0, p1, w1s, w1n, b1):
    return pl.pallas_call(
        _tc1_body,
        grid=(N // BR,),
        in_specs=[
            pl.BlockSpec((BR, D), lambda i: (i, 0)),
            pl.BlockSpec((BR, WPAD), lambda i: (i, 0)),
            pl.BlockSpec((BR, WPAD), lambda i: (i, 0)),
            pl.BlockSpec((D, H), lambda i: (0, 0)),
            pl.BlockSpec((D, H), lambda i: (0, 0)),
            pl.BlockSpec((1, H), lambda i: (0, 0)),
        ],
        out_specs=[
            pl.BlockSpec((BR, H), lambda i: (i, 0)),
            pl.BlockSpec((BR, 1), lambda i: (i, 0)),
        ],
        out_shape=[
            jax.ShapeDtypeStruct((N, H), jnp.float32),
            jax.ShapeDtypeStruct((N, 1), jnp.float32),
        ],
    )(x, p0, p1, w1s, w1n, b1)


def _tc2_body(h1, q0, q1, deg, w2s, w2n, b2, we1a, we1b,
              wb1, bb1, wb2, bb2, wk1, bk1, wk2, bk2,
              psrc, pdst, bx, kc, ks):
    agg = (q0[...] + q1[...]) / deg[...]
    h2 = jnp.maximum(h1[...] @ w2s[...] + agg @ w2n[...] + b2[...], 0.0)
    psrc[...] = h2 @ we1a[...]
    pdst[...] = h2 @ we1b[...]
    tb = jnp.maximum(h2 @ wb1[...] + bb1[...], 0.0)
    bx[...] = tb @ wb2[...] + bb2[...]
    tk = jnp.maximum(h2 @ wk1[...] + bk1[...], 0.0)
    kl = tk @ wk2[...] + bk2[...]
    kcv = 1.0 + 7.0 * jax.nn.sigmoid(kl)
    kd = jnp.clip(jnp.round(kcv), 1.0, 8.0)
    kc[...] = kcv
    ks[...] = kcv + (kd - kcv)


def _tc2(h1, q0, q1, deg, pr):
    full = lambda a, b: pl.BlockSpec((a, b), lambda i: (0, 0))
    row = lambda b: pl.BlockSpec((BR, b), lambda i: (i, 0))
    return pl.pallas_call(
        _tc2_body,
        grid=(N // BR,),
        in_specs=[
            row(H), row(H), row(H), row(1),
            full(H, H), full(H, H), full(1, H),
            full(H, 64), full(H, 64),
            full(H, 64), full(1, 64), full(64, 2), full(1, 2),
            full(H, 32), full(1, 32), full(32, 1), full(1, 1),
        ],
        out_specs=[row(64), row(64), row(2), row(1), row(1)],
        out_shape=[
            jax.ShapeDtypeStruct((N, 64), jnp.float32),
            jax.ShapeDtypeStruct((N, 64), jnp.float32),
            jax.ShapeDtypeStruct((N, 2), jnp.float32),
            jax.ShapeDtypeStruct((N, 1), jnp.float32),
            jax.ShapeDtypeStruct((N, 1), jnp.float32),
        ],
    )(h1, q0, q1, deg, pr['W2s'], pr['W2n'], pr['b2'].reshape(1, H),
      pr['We1'][:H], pr['We1'][H:2 * H],
      pr['Wb1'], pr['bb1'].reshape(1, 64), pr['Wb2'], pr['bb2'].reshape(1, 2),
      pr['Wk1'], pr['bk1'].reshape(1, 32), pr['Wk2'], pr['bk2'].reshape(1, 1))


def kernel(x, edge_index, edge_weight, params):
    src = edge_index[0].astype(jnp.int32)
    dst = edge_index[1].astype(jnp.int32)
    xpad = jnp.concatenate(
        [x, jnp.ones((N, 1), jnp.float32), jnp.zeros((N, WPAD - D - 1), jnp.float32)],
        axis=1)

    p0, p1 = _sc_scatter_144(xpad, src, dst)
    h1, deg = _tc1(x, p0, p1, params['W1s'], params['W1n'],
                   params['b1'].reshape(1, H))
    q0, q1 = _sc_scatter_128(h1, src, dst)
    psrc, pdst, bx, kc, ks = _tc2(h1, q0, q1, deg, params)

    par = jnp.concatenate([
        params['We1'][2 * H],
        params['be1'],
        params['We2'][:, 0],
        jnp.broadcast_to(params['be2'], (64,)),
    ]).astype(jnp.float32)
    logits = _sc_edge(psrc, pdst, src, dst, edge_weight, par)

    return (logits, bx, kc[:, 0], ks[:, 0])
